# baseline (speedup 1.0000x reference)
import functools
import math

import jax
import jax.numpy as jnp
from jax import lax
from jax.experimental import pallas as pl
from jax.experimental.pallas import tpu as pltpu
from jax.experimental.pallas import tpu_sc as plsc

D_MODEL = 512
MAX_SEQ = 200
VOCAB = 76
VOCAB_PAD = 80
BATCH = 1024

_info = plsc.get_sparse_core_info()
_NUM_CORES = _info.num_cores
_NUM_SUBCORES = _info.num_subcores
_NUM_WORKERS = _NUM_CORES * _NUM_SUBCORES
_LANES = _info.num_lanes

N_ROWS = BATCH * MAX_SEQ
ROWS_PER_W = N_ROWS // _NUM_WORKERS
CHUNK = 40
N_CHUNKS = ROWS_PER_W // CHUNK
NBUF = 4
GROUPS = D_MODEL // 32


def _positional_encoding():
    even_i = jnp.arange(0, D_MODEL, 2, dtype=jnp.float32)
    denominator = jnp.power(10000.0, even_i / D_MODEL)
    position = jnp.arange(0, MAX_SEQ, 1, dtype=jnp.float32).reshape(MAX_SEQ, 1)
    even_pe = jnp.sin(position / denominator)
    odd_pe = jnp.cos(position / denominator)
    return jnp.stack([even_pe, odd_pe], axis=2).reshape(MAX_SEQ, D_MODEL)


def _fuse_body(table_ref, pe_ref, out_ref):
    out_ref[...] = (
        table_ref[...] * math.sqrt(float(D_MODEL)) + pe_ref[...][:, None, :]
    )


_build_fused = pl.pallas_call(
    _fuse_body,
    out_shape=jax.ShapeDtypeStruct((MAX_SEQ, VOCAB_PAD, D_MODEL), jnp.float32),
)

_mesh = plsc.VectorSubcoreMesh(core_axis_name="c", subcore_axis_name="s")


@functools.partial(
    pl.kernel,
    out_type=jax.ShapeDtypeStruct((N_ROWS, D_MODEL), jnp.float32),
    mesh=_mesh,
    scratch_types=[
        pltpu.VMEM((ROWS_PER_W,), jnp.int32),
        [pltpu.VMEM((CHUNK, D_MODEL // 2), jnp.int32) for _ in range(NBUF)],
        [pltpu.VMEM((CHUNK, D_MODEL), jnp.float32) for _ in range(NBUF)],
        [pltpu.SemaphoreType.DMA for _ in range(NBUF)],
        [pltpu.SemaphoreType.DMA for _ in range(NBUF)],
    ],
)
def _gather_kernel(tok_hbm, fused_hbm, out_hbm, idx_v, bbufs, fbufs, gsems, ssems):
    wid = lax.axis_index("s") * _NUM_CORES + lax.axis_index("c")
    base = wid * ROWS_PER_W
    pltpu.sync_copy(tok_hbm.at[pl.ds(base, ROWS_PER_W)], idx_v)

    lanes = lax.iota(jnp.int32, _LANES)

    def idx_body(j, carry):
        o = j * _LANES
        tok = idx_v[pl.ds(o, _LANES)]
        pos = jnp.remainder(o + lanes, MAX_SEQ)
        idx_v[pl.ds(o, _LANES)] = pos * VOCAB_PAD + tok
        return carry

    lax.fori_loop(0, ROWS_PER_W // _LANES, idx_body, 0)

    def fire_gather(c, b):
        pltpu.async_copy(
            fused_hbm.at[idx_v.at[pl.ds(c * CHUNK, CHUNK)]],
            bbufs[b],
            gsems[b],
        )

    def wait_gather(b):
        pltpu.make_async_copy(
            fused_hbm.at[pl.ds(0, CHUNK)],
            bbufs[b],
            gsems[b],
        ).wait()

    def fire_scatter(c, b):
        pltpu.async_copy(
            fbufs[b], out_hbm.at[pl.ds(base + c * CHUNK, CHUNK)], ssems[b]
        )

    def wait_scatter(b):
        pltpu.make_async_copy(
            fbufs[b], out_hbm.at[pl.ds(base, CHUNK)], ssems[b]
        ).wait()

    def widen(b):
        def row_body(r, carry):
            for k in range(GROUPS):
                v = bbufs[b][r, pl.ds(_LANES * k, _LANES)]
                lo = lax.bitcast_convert_type(v << 16, jnp.float32)
                hi = lax.bitcast_convert_type(v & jnp.int32(-65536), jnp.float32)
                fbufs[b][r, pl.ds(32 * k, _LANES)] = lo
                fbufs[b][r, pl.ds(32 * k + 16, _LANES)] = hi
            return carry

        lax.fori_loop(0, CHUNK, row_body, 0)

    fire_gather(0, 0)
    fire_gather(1, 1)
    for c in range(4):
        wait_gather(c)
        widen(c)
        fire_scatter(c, c)
        fire_gather(c + 2, (c + 2) % NBUF)

    def chunk_body(g, carry):
        for k in range(NBUF):
            c = 4 + g * NBUF + k
            wait_gather(k)
            wait_scatter(k)
            widen(k)
            fire_scatter(c, k)
            fire_gather(c + 2, (k + 2) % NBUF)
        return carry

    lax.fori_loop(0, (N_CHUNKS - 8) // NBUF, chunk_body, 0)

    for c in range(N_CHUNKS - 4, N_CHUNKS):
        b = c % NBUF
        wait_gather(b)
        wait_scatter(b)
        widen(b)
        fire_scatter(c, b)
        if c + 2 < N_CHUNKS:
            fire_gather(c + 2, (c + 2) % NBUF)
    for b in range(NBUF):
        wait_scatter(b)


def kernel(token_ids, embedding_table):
    tok_flat = token_ids.reshape(-1).astype(jnp.int32)
    table_pad = jnp.pad(embedding_table, ((0, VOCAB_PAD - VOCAB), (0, 0)))
    fused = _build_fused(table_pad, _positional_encoding())
    fused_i32 = jax.lax.bitcast_convert_type(
        fused.astype(jnp.bfloat16)
        .reshape(MAX_SEQ * VOCAB_PAD, GROUPS, 2, _LANES)
        .transpose(0, 1, 3, 2),
        jnp.int32,
    ).reshape(MAX_SEQ * VOCAB_PAD, D_MODEL // 2)
    out = _gather_kernel(tok_flat, fused_i32)
    return out.reshape(BATCH, MAX_SEQ, D_MODEL)


# speedup vs baseline: 1.9519x; 1.9519x over previous
import functools
import math

import jax
import jax.numpy as jnp
from jax import lax
from jax.experimental import pallas as pl
from jax.experimental.pallas import tpu as pltpu
from jax.experimental.pallas import tpu_sc as plsc

D_MODEL = 512
MAX_SEQ = 200
VOCAB = 76
VOCAB_PAD = 80
BATCH = 1024

_info = plsc.get_sparse_core_info()
_NUM_CORES = _info.num_cores
_NUM_SUBCORES = _info.num_subcores
_NUM_WORKERS = _NUM_CORES * _NUM_SUBCORES
_LANES = _info.num_lanes

N_ROWS = BATCH * MAX_SEQ
ROWS_PER_W = N_ROWS // _NUM_WORKERS
CHUNK = 40
N_CHUNKS = ROWS_PER_W // CHUNK
NBUF = 5


def _positional_encoding():
    even_i = jnp.arange(0, D_MODEL, 2, dtype=jnp.float32)
    denominator = jnp.power(10000.0, even_i / D_MODEL)
    position = jnp.arange(0, MAX_SEQ, 1, dtype=jnp.float32).reshape(MAX_SEQ, 1)
    even_pe = jnp.sin(position / denominator)
    odd_pe = jnp.cos(position / denominator)
    return jnp.stack([even_pe, odd_pe], axis=2).reshape(MAX_SEQ, D_MODEL)


def _fuse_body(table_ref, pe_ref, out_ref):
    out_ref[...] = (
        table_ref[...] * math.sqrt(float(D_MODEL)) + pe_ref[...][:, None, :]
    )


_build_fused = pl.pallas_call(
    _fuse_body,
    out_shape=jax.ShapeDtypeStruct((MAX_SEQ, VOCAB_PAD, D_MODEL), jnp.float32),
)

_mesh = plsc.VectorSubcoreMesh(core_axis_name="c", subcore_axis_name="s")


@functools.partial(
    pl.kernel,
    out_type=jax.ShapeDtypeStruct((N_ROWS, D_MODEL), jnp.float32),
    mesh=_mesh,
    scratch_types=[
        pltpu.VMEM((ROWS_PER_W,), jnp.int32),
        [pltpu.VMEM((CHUNK, D_MODEL), jnp.float32) for _ in range(NBUF)],
        [pltpu.SemaphoreType.DMA for _ in range(NBUF)],
        [pltpu.SemaphoreType.DMA for _ in range(NBUF)],
    ],
)
def _gather_kernel(tok_hbm, fused_hbm, out_hbm, idx_v, bufs, gsems, ssems):
    wid = lax.axis_index("s") * _NUM_CORES + lax.axis_index("c")
    base = wid * ROWS_PER_W
    pltpu.sync_copy(tok_hbm.at[pl.ds(base, ROWS_PER_W)], idx_v)

    lanes = lax.iota(jnp.int32, _LANES)

    def idx_body(j, carry):
        o = j * _LANES
        tok = idx_v[pl.ds(o, _LANES)]
        pos = jnp.remainder(o + lanes, MAX_SEQ)
        idx_v[pl.ds(o, _LANES)] = pos * VOCAB_PAD + tok
        return carry

    lax.fori_loop(0, ROWS_PER_W // _LANES, idx_body, 0)

    def fire_gather(c, b):
        pltpu.async_copy(
            fused_hbm.at[idx_v.at[pl.ds(c * CHUNK, CHUNK)]], bufs[b], gsems[b]
        )

    def wait_gather(b):
        pltpu.make_async_copy(
            out_hbm.at[pl.ds(base, CHUNK)], bufs[b], gsems[b]
        ).wait()

    def fire_scatter(c, b):
        pltpu.async_copy(
            bufs[b], out_hbm.at[pl.ds(base + c * CHUNK, CHUNK)], ssems[b]
        )

    def wait_scatter(b):
        pltpu.make_async_copy(
            bufs[b], out_hbm.at[pl.ds(base, CHUNK)], ssems[b]
        ).wait()

    fire_gather(0, 0)
    fire_gather(1, 1)
    for c in (0, 1, 2):
        wait_gather(c)
        fire_scatter(c, c)
        fire_gather(c + 2, c + 2)

    def chunk_body(g, carry):
        for k in range(NBUF):
            c = 3 + g * NBUF + k
            b = (3 + k) % NBUF
            b2 = (5 + k) % NBUF
            wait_gather(b)
            fire_scatter(c, b)
            wait_scatter(b2)
            fire_gather(c + 2, b2)
        return carry

    lax.fori_loop(0, (N_CHUNKS - 5) // NBUF, chunk_body, 0)

    for c in (N_CHUNKS - 2, N_CHUNKS - 1):
        b = c % NBUF
        wait_gather(b)
        fire_scatter(c, b)
    for b in range(NBUF):
        wait_scatter(b)


def kernel(token_ids, embedding_table):
    tok_flat = token_ids.reshape(-1).astype(jnp.int32)
    table_pad = jnp.pad(embedding_table, ((0, VOCAB_PAD - VOCAB), (0, 0)))
    fused = _build_fused(table_pad, _positional_encoding()).reshape(
        MAX_SEQ * VOCAB_PAD, D_MODEL
    )
    out = _gather_kernel(tok_flat, fused)
    return out.reshape(BATCH, MAX_SEQ, D_MODEL)
